# Initial kernel scaffold; baseline (speedup 1.0000x reference)
import jax
import jax.numpy as jnp
from jax.experimental import pallas as pl


def kernel(x, edge_index, W_self, W_nbr, b):
    raise NotImplementedError("write your pallas kernel here")


# speedup vs baseline: 3.9720x; 3.9720x over previous
import functools

import jax
import jax.numpy as jnp
from jax import lax
from jax.experimental import pallas as pl
from jax.experimental.pallas import tpu as pltpu
from jax.experimental.pallas import tpu_sc as plsc

_N = 10000
_D = 128
_C = 128
_NC = 2
_NS = 16
_NW = _NC * _NS
_KPS = 5
_RPT = _KPS * _C
_NPAD = _NS * _RPT


def _zero_fill(buf):
  zero16 = jnp.zeros((16,), jnp.float32)

  @pl.loop(0, _C)
  def _(i):
    for j in range(_D // 16):
      buf[i, pl.ds(j * 16, 16)] = zero16


def _sc_scatter_kernel(gather_rows):
  def kfn(x_hbm, src_hbm, dst_hbm, out_hbm,
          src_v, dst_v, rows_v, work_v, agg_sh, sem):
    cid = lax.axis_index("c")
    sid = lax.axis_index("s")
    wid = sid * _NC + cid
    ept = src_hbm.shape[0] // _NW
    chunks = ept // _C
    base = wid * ept
    rbase = sid * _RPT

    _zero_fill(rows_v)
    for k in range(_KPS):
      pltpu.sync_copy(rows_v, agg_sh.at[pl.ds(rbase + k * _C, _C)])

    if not gather_rows:
      one16 = jnp.ones((16,), jnp.float32)

      @pl.loop(0, _C)
      def _(i):
        for j in range(_D // 16):
          work_v[i, pl.ds(j * 16, 16)] = one16

    plsc.subcore_barrier()

    @pl.loop(0, chunks)
    def _(g):
      off = base + g * _C
      pltpu.sync_copy(dst_hbm.at[pl.ds(off, _C)], dst_v)
      if gather_rows:
        pltpu.sync_copy(src_hbm.at[pl.ds(off, _C)], src_v)
        pltpu.async_copy(x_hbm.at[src_v], work_v, sem).wait()
      pltpu.sync_copy(work_v, agg_sh.at[dst_v], add=True)

    plsc.subcore_barrier()

    for k in range(_KPS):
      r = rbase + k * _C
      pltpu.sync_copy(agg_sh.at[pl.ds(r, _C)], rows_v)
      pltpu.sync_copy(rows_v, out_hbm.at[cid, pl.ds(r, _C)])

  return kfn


def _sc_run(x, src, dst, gather_rows):
  mesh = plsc.VectorSubcoreMesh(core_axis_name="c", subcore_axis_name="s")
  return functools.partial(
      pl.kernel,
      mesh=mesh,
      out_type=jax.ShapeDtypeStruct((_NC, _NPAD, _D), jnp.float32),
      scratch_types=[
          pltpu.VMEM((_C,), jnp.int32),
          pltpu.VMEM((_C,), jnp.int32),
          pltpu.VMEM((_C, _D), jnp.float32),
          pltpu.VMEM((_C, _D), jnp.float32),
          pltpu.VMEM_SHARED((_NPAD, _D), jnp.float32),
          pltpu.SemaphoreType.DMA,
      ],
  )(_sc_scatter_kernel(gather_rows))(x, src, dst)


def _combine(x, w_self, w_nbr, b2d, agg, deg):
  rows = 1000
  grid = (_N // rows,)

  def body(x_ref, ws_ref, wn_ref, b_ref, a_ref, d_ref, o_ref):
    a = a_ref[0] + a_ref[1]
    d = d_ref[0][:, 0:1] + d_ref[1][:, 0:1]
    a = a / jnp.maximum(d, 1.0)
    out = (jnp.dot(x_ref[...], ws_ref[...], preferred_element_type=jnp.float32)
           + jnp.dot(a, wn_ref[...], preferred_element_type=jnp.float32)
           + b_ref[...])
    o_ref[...] = jnp.maximum(out, 0.0)

  return pl.pallas_call(
      body,
      grid=grid,
      in_specs=[
          pl.BlockSpec((rows, _D), lambda i: (i, 0)),
          pl.BlockSpec((_D, _D), lambda i: (0, 0)),
          pl.BlockSpec((_D, _D), lambda i: (0, 0)),
          pl.BlockSpec((1, _D), lambda i: (0, 0)),
          pl.BlockSpec((_NC, rows, _D), lambda i: (0, i, 0)),
          pl.BlockSpec((_NC, rows, _D), lambda i: (0, i, 0)),
      ],
      out_specs=pl.BlockSpec((rows, _D), lambda i: (i, 0)),
      out_shape=jax.ShapeDtypeStruct((_N, _D), jnp.float32),
  )(x, w_self, w_nbr, b2d, agg, deg)


def kernel(x, edge_index, W_self, W_nbr, b):
  src = edge_index[0]
  dst = edge_index[1]
  e = src.shape[0]
  ept = -(-e // (_NW * _C)) * _C
  epad = ept * _NW - e
  src = jnp.concatenate([src, jnp.zeros((epad,), jnp.int32)])
  fill = _N + (jnp.arange(epad, dtype=jnp.int32) % (_NPAD - _N))
  dst = jnp.concatenate([dst, fill])
  agg = _sc_run(x, src, dst, gather_rows=True)
  deg = _sc_run(x, src, dst, gather_rows=False)
  return _combine(x, W_self, W_nbr, b.reshape(1, _D), agg, deg)
